# baseline (speedup 1.0000x reference)
import functools

import jax
import jax.numpy as jnp
from jax.experimental import pallas as pl
from jax.experimental.pallas import tpu as pltpu

_INPUT = 1024
_OUTPUT = 1024
_EXPERTS = 8
_EPG = 2


def _moe_kernel(x_ref, gw_ref, ew_ref, eb_ref, y_ref, bl_ref, sf_ref):
    g = pl.program_id(0)
    n = x_ref.shape[0]

    @pl.when(g == 0)
    def _gate():
        xt = x_ref[...]
        logits = jax.lax.dot_general(
            xt, gw_ref[...], (((1,), (1,)), ((), ())),
            preferred_element_type=jnp.float32)
        iota = jax.lax.broadcasted_iota(jnp.int32, (n, _EXPERTS), 1)
        m1 = jnp.max(logits, axis=1, keepdims=True)
        i1 = jnp.min(jnp.where(logits == m1, iota, _EXPERTS), axis=1,
                     keepdims=True)
        l2 = jnp.where(iota == i1, -jnp.inf, logits)
        m2 = jnp.max(l2, axis=1, keepdims=True)
        i2 = jnp.min(jnp.where(l2 == m2, iota, _EXPERTS), axis=1,
                     keepdims=True)
        ex = jnp.exp(m2 - m1)
        denom = 1.0 + ex
        s1 = 1.0 / denom
        s2 = ex / denom
        sf = jnp.where(iota == i1, s1, 0.0) + jnp.where(iota == i2, s2, 0.0)
        sf_ref[...] = sf

        def cv(v):
            mean = jnp.sum(v) / _EXPERTS
            var = jnp.sum((v - mean) ** 2) / (_EXPERTS - 1)
            return var / (mean * mean + 1e-10)

        imp = jnp.sum(sf, axis=0)
        load = jnp.sum((sf > 0.0).astype(jnp.float32), axis=0)
        bl_ref[...] = jnp.reshape(0.01 * (cv(imp) + cv(load)), (1, 1))

        y_ref[...] = jax.lax.dot_general(
            sf, eb_ref[...], (((1,), (0,)), ((), ())),
            preferred_element_type=jnp.float32)

    xb = x_ref[...].astype(jnp.bfloat16)
    iota = jax.lax.broadcasted_iota(jnp.int32, (n, _EXPERTS), 1)
    sf = sf_ref[...]
    acc = y_ref[...]
    for j in range(_EPG):
        pe = jax.lax.dot_general(
            xb, ew_ref[j], (((1,), (1,)), ((), ())),
            preferred_element_type=jnp.float32)
        sf_col = jnp.sum(jnp.where(iota == g * _EPG + j, sf, 0.0), axis=1,
                         keepdims=True)
        acc = acc + sf_col * pe
    y_ref[...] = acc


@functools.partial(jax.jit, static_argnames=("interpret",))
def _run(x, gate_W, expert_W, expert_b, interpret=False):
    n = x.size // x.shape[-1]
    xf = x.reshape(n, _INPUT)
    y, bl = pl.pallas_call(
        _moe_kernel,
        grid=(_EXPERTS // _EPG,),
        in_specs=[
            pl.BlockSpec((n, _INPUT), lambda g: (0, 0)),
            pl.BlockSpec((_EXPERTS, _INPUT), lambda g: (0, 0)),
            pl.BlockSpec((_EPG, _OUTPUT, _INPUT), lambda g: (g, 0, 0)),
            pl.BlockSpec((_EXPERTS, _OUTPUT), lambda g: (0, 0)),
        ],
        out_specs=[
            pl.BlockSpec((n, _OUTPUT), lambda g: (0, 0)),
            pl.BlockSpec((1, 1), lambda g: (0, 0)),
        ],
        out_shape=[
            jax.ShapeDtypeStruct((n, _OUTPUT), jnp.float32),
            jax.ShapeDtypeStruct((1, 1), jnp.float32),
        ],
        scratch_shapes=[
            pltpu.VMEM((n, _EXPERTS), jnp.float32),
        ],
        interpret=interpret,
    )(xf, gate_W, expert_W, expert_b)
    return y.reshape(x.shape[:-1] + (_OUTPUT,)), bl[0, 0]


def kernel(x, gate_W, expert_W, expert_b):
    return _run(x, gate_W, expert_W, expert_b)


# speedup vs baseline: 1.0386x; 1.0386x over previous
import functools

import jax
import jax.numpy as jnp
from jax.experimental import pallas as pl
from jax.experimental.pallas import tpu as pltpu

_INPUT = 1024
_OUTPUT = 1024
_EXPERTS = 8
_EPG = 1


def _moe_kernel(x_ref, gw_ref, ew_ref, eb_ref, y_ref, bl_ref, sf_ref):
    g = pl.program_id(0)
    n = x_ref.shape[0]

    @pl.when(g == 0)
    def _gate():
        xt = x_ref[...]
        logits = jax.lax.dot_general(
            xt, gw_ref[...], (((1,), (1,)), ((), ())),
            preferred_element_type=jnp.float32)
        iota = jax.lax.broadcasted_iota(jnp.int32, (n, _EXPERTS), 1)
        m1 = jnp.max(logits, axis=1, keepdims=True)
        i1 = jnp.min(jnp.where(logits == m1, iota, _EXPERTS), axis=1,
                     keepdims=True)
        l2 = jnp.where(iota == i1, -jnp.inf, logits)
        m2 = jnp.max(l2, axis=1, keepdims=True)
        i2 = jnp.min(jnp.where(l2 == m2, iota, _EXPERTS), axis=1,
                     keepdims=True)
        ex = jnp.exp(m2 - m1)
        denom = 1.0 + ex
        s1 = 1.0 / denom
        s2 = ex / denom
        sf = jnp.where(iota == i1, s1, 0.0) + jnp.where(iota == i2, s2, 0.0)
        sf_ref[...] = sf

        def cv(v):
            mean = jnp.sum(v) / _EXPERTS
            var = jnp.sum((v - mean) ** 2) / (_EXPERTS - 1)
            return var / (mean * mean + 1e-10)

        imp = jnp.sum(sf, axis=0)
        load = jnp.sum((sf > 0.0).astype(jnp.float32), axis=0)
        bl_ref[...] = jnp.reshape(0.01 * (cv(imp) + cv(load)), (1, 1))

        y_ref[...] = jax.lax.dot_general(
            sf, eb_ref[...], (((1,), (0,)), ((), ())),
            preferred_element_type=jnp.float32)

    xb = x_ref[...].astype(jnp.bfloat16)
    iota = jax.lax.broadcasted_iota(jnp.int32, (n, _EXPERTS), 1)
    sf = sf_ref[...]
    acc = y_ref[...]
    for j in range(_EPG):
        pe = jax.lax.dot_general(
            xb, ew_ref[j], (((1,), (1,)), ((), ())),
            preferred_element_type=jnp.float32)
        sf_col = jnp.sum(jnp.where(iota == g * _EPG + j, sf, 0.0), axis=1,
                         keepdims=True)
        acc = acc + sf_col * pe
    y_ref[...] = acc


@functools.partial(jax.jit, static_argnames=("interpret",))
def _run(x, gate_W, expert_W, expert_b, interpret=False):
    n = x.size // x.shape[-1]
    xf = x.reshape(n, _INPUT)
    y, bl = pl.pallas_call(
        _moe_kernel,
        grid=(_EXPERTS // _EPG,),
        in_specs=[
            pl.BlockSpec((n, _INPUT), lambda g: (0, 0)),
            pl.BlockSpec((_EXPERTS, _INPUT), lambda g: (0, 0)),
            pl.BlockSpec((_EPG, _OUTPUT, _INPUT), lambda g: (g, 0, 0)),
            pl.BlockSpec((_EXPERTS, _OUTPUT), lambda g: (0, 0)),
        ],
        out_specs=[
            pl.BlockSpec((n, _OUTPUT), lambda g: (0, 0)),
            pl.BlockSpec((1, 1), lambda g: (0, 0)),
        ],
        out_shape=[
            jax.ShapeDtypeStruct((n, _OUTPUT), jnp.float32),
            jax.ShapeDtypeStruct((1, 1), jnp.float32),
        ],
        scratch_shapes=[
            pltpu.VMEM((n, _EXPERTS), jnp.float32),
        ],
        interpret=interpret,
    )(xf, gate_W, expert_W, expert_b)
    return y.reshape(x.shape[:-1] + (_OUTPUT,)), bl[0, 0]


def kernel(x, gate_W, expert_W, expert_b):
    return _run(x, gate_W, expert_W, expert_b)
